# Initial kernel scaffold; baseline (speedup 1.0000x reference)
import jax
import jax.numpy as jnp
from jax.experimental import pallas as pl


def kernel(x, stc_enc, edge_index, batch, W_emb, b_emb, W1, b1, Ws1, bs1, W2, b2, Ws2, bs2, Whp_W, Whp_b, Wc, bc):
    raise NotImplementedError("write your pallas kernel here")


# speedup vs baseline: 16.0920x; 16.0920x over previous
import functools

import jax
import jax.numpy as jnp
from jax import lax
from jax.experimental import pallas as pl
from jax.experimental.pallas import tpu as pltpu
from jax.experimental.pallas import tpu_sc as plsc

_NC = 2
_NS = 16
_CHUNK = 80
_ROWS = 1024


def _degree_call(dstd, init):
    nc, ns, nchunk, chunk = dstd.shape
    n = init.shape[1]
    rpt = n // ns
    mesh = plsc.VectorSubcoreMesh(core_axis_name="c", subcore_axis_name="s")

    @functools.partial(
        pl.kernel,
        out_type=jax.ShapeDtypeStruct((nc, n, 16), jnp.float32),
        mesh=mesh,
        scratch_types=[
            pltpu.VMEM((nchunk, chunk), jnp.int32),
            pltpu.VMEM((chunk, 16), jnp.float32),
            pltpu.VMEM_SHARED((n, 16), jnp.float32),
        ],
    )
    def deg_k(dst_hbm, init_hbm, out_hbm, dstv, onesv, acc):
        c = lax.axis_index("c")
        s = lax.axis_index("s")
        r0 = s * rpt
        pltpu.sync_copy(init_hbm.at[c, pl.ds(r0, rpt)], acc.at[pl.ds(r0, rpt)])
        pltpu.sync_copy(init_hbm.at[0, pl.ds(0, chunk)], onesv)
        pltpu.sync_copy(dst_hbm.at[c, s], dstv)
        plsc.subcore_barrier()

        def body(j, carry):
            pltpu.sync_copy(onesv, acc.at[dstv.at[j]], add=True)
            return carry

        lax.fori_loop(0, nchunk, body, 0)
        plsc.subcore_barrier()
        pltpu.sync_copy(acc.at[pl.ds(r0, rpt)], out_hbm.at[c, pl.ds(r0, rpt)])

    return deg_k(dstd, init)


def _agg_call(table, srcx, dsta):
    nt, ns, nchunk, chunk = srcx.shape
    n = table.shape[0] // nt
    rpt = n // ns
    npass = nt // _NC
    mesh = plsc.VectorSubcoreMesh(core_axis_name="c", subcore_axis_name="s")

    @functools.partial(
        pl.kernel,
        out_type=jax.ShapeDtypeStruct((nt, n, 64), jnp.float32),
        mesh=mesh,
        scratch_types=[
            pltpu.VMEM((nchunk, chunk), jnp.int32),
            pltpu.VMEM((nchunk, chunk), jnp.int32),
            pltpu.VMEM((chunk, 64), jnp.float32),
            pltpu.VMEM((chunk, 64), jnp.float32),
            pltpu.SemaphoreType.DMA,
            pltpu.SemaphoreType.DMA,
            pltpu.VMEM_SHARED((n, 64), jnp.float32),
        ],
        compiler_params=pltpu.CompilerParams(use_tc_tiling_on_sc=False),
    )
    def agg_k(table_hbm, src_hbm, dst_hbm, out_hbm,
              srcv, dstv, rows0, rows1, sem0, sem1, acc):
        c = lax.axis_index("c")
        s = lax.axis_index("s")
        r0 = s * rpt
        pltpu.sync_copy(dst_hbm.at[s], dstv)

        rows = (rows0, rows1)
        sems = (sem0, sem1)

        def start(j, b):
            pltpu.async_copy(table_hbm.at[srcv.at[j]], rows[b], sems[b])

        def wait(b):
            pltpu.make_async_copy(table_hbm.at[srcv.at[0]],
                                  rows[b], sems[b]).wait()

        def half(j, b):
            @pl.when(j + 1 < nchunk)
            def _():
                start(j + 1, 1 - b)
            wait(b)
            pltpu.sync_copy(rows[b], acc.at[dstv.at[j]], add=True)

        def body(j2, carry):
            half(2 * j2, 0)
            half(2 * j2 + 1, 1)
            return carry

        for p in range(npass):
            t = npass * c + p
            pltpu.sync_copy(table_hbm.at[pl.ds(t * n + r0, rpt)],
                            acc.at[pl.ds(r0, rpt)])
            pltpu.sync_copy(src_hbm.at[t, s], srcv)
            plsc.subcore_barrier()
            start(0, 0)
            lax.fori_loop(0, nchunk // 2, body, 0)
            plsc.subcore_barrier()
            pltpu.sync_copy(acc.at[pl.ds(r0, rpt)],
                            out_hbm.at[t, pl.ds(r0, rpt)])

    return agg_k(table, srcx, dsta)


def _dis(dc_ref):
    deg = dc_ref[0] + dc_ref[1]
    return lax.rsqrt(deg[:, 0:1])


def _split_store(out_ref, gh, gs):
    out_ref[0] = gh[:, :64]
    out_ref[1] = gh[:, 64:]
    out_ref[2] = gs[:, :64]
    out_ref[3] = gs[:, 64:]


def _join(a_ref):
    ah = jnp.concatenate([a_ref[0], a_ref[1]], axis=1)
    as_ = jnp.concatenate([a_ref[2], a_ref[3]], axis=1)
    return ah, as_


def _dense1_body(x_ref, stc_ref, dc_ref, wemb_ref, bemb_ref,
                 w1a_ref, w1b_ref, ws1_ref, out_ref):
    dis = _dis(dc_ref)
    s = jnp.dot(stc_ref[...], wemb_ref[...],
                preferred_element_type=jnp.float32) + bemb_ref[...]
    gh = (jnp.dot(x_ref[...], w1a_ref[...], preferred_element_type=jnp.float32)
          + jnp.dot(s, w1b_ref[...], preferred_element_type=jnp.float32)) * dis
    gs = jnp.dot(s, ws1_ref[...], preferred_element_type=jnp.float32) * dis
    _split_store(out_ref, gh, gs)


def _dense2_body(a_ref, dc_ref, b1_ref, bs1_ref,
                 w2a_ref, w2b_ref, ws2_ref, out_ref):
    dis = _dis(dc_ref)
    ah, as_ = _join(a_ref)
    h1 = jax.nn.relu(ah * dis + b1_ref[...])
    s1 = jnp.tanh(as_ * dis + bs1_ref[...])
    gh = (jnp.dot(h1, w2a_ref[...], preferred_element_type=jnp.float32)
          + jnp.dot(s1, w2b_ref[...], preferred_element_type=jnp.float32)) * dis
    gs = jnp.dot(s1, ws2_ref[...], preferred_element_type=jnp.float32) * dis
    _split_store(out_ref, gh, gs)


def _dense3_body(a_ref, dc_ref, b2_ref, bs2_ref, wpa_ref, wpb_ref, bhp_ref,
                 wc_ref, bc_ref, out_ref, *, nclass):
    dis = _dis(dc_ref)
    ah, as_ = _join(a_ref)
    h2 = jax.nn.relu(ah * dis + b2_ref[...])
    s2 = jnp.tanh(as_ * dis + bs2_ref[...])
    hp = jax.nn.relu(
        jnp.dot(h2, wpa_ref[...], preferred_element_type=jnp.float32)
        + jnp.dot(s2, wpb_ref[...], preferred_element_type=jnp.float32)
        + bhp_ref[...])
    logits = jnp.dot(hp, wc_ref[...],
                     preferred_element_type=jnp.float32) + bc_ref[...]
    col = lax.broadcasted_iota(jnp.int32, logits.shape, 1)
    valid = col < nclass
    mx = jnp.max(jnp.where(valid, logits, -3e38), axis=1, keepdims=True)
    ex = jnp.where(valid, jnp.exp(logits - mx), 0.0)
    lse = jnp.log(jnp.sum(ex, axis=1, keepdims=True))
    out_ref[...] = logits - mx - lse


def _row_spec(r, width):
    return pl.BlockSpec((r, width), lambda i: (i, 0))


def _full_spec(shape):
    return pl.BlockSpec(shape, lambda i: tuple(0 for _ in shape))


def _stacked_spec(r, width, planes=2):
    return pl.BlockSpec((planes, r, width), lambda i: (0, i, 0))


def _dense1(x, stc, degc, wemb, bemb, w1a, w1b, ws1, n, r):
    return pl.pallas_call(
        _dense1_body,
        grid=(n // r,),
        in_specs=[
            _row_spec(r, x.shape[1]),
            _row_spec(r, stc.shape[1]),
            _stacked_spec(r, 16),
            _full_spec(wemb.shape), _full_spec(bemb.shape),
            _full_spec(w1a.shape), _full_spec(w1b.shape), _full_spec(ws1.shape),
        ],
        out_specs=_stacked_spec(r, 64, 4),
        out_shape=jax.ShapeDtypeStruct((4, n, 64), jnp.float32),
    )(x, stc, degc, wemb, bemb, w1a, w1b, ws1)


def _dense2(a, degc, b1, bs1, w2a, w2b, ws2, n, r):
    return pl.pallas_call(
        _dense2_body,
        grid=(n // r,),
        in_specs=[
            _stacked_spec(r, 64, 4),
            _stacked_spec(r, 16),
            _full_spec(b1.shape), _full_spec(bs1.shape),
            _full_spec(w2a.shape), _full_spec(w2b.shape), _full_spec(ws2.shape),
        ],
        out_specs=_stacked_spec(r, 64, 4),
        out_shape=jax.ShapeDtypeStruct((4, n, 64), jnp.float32),
    )(a, degc, b1, bs1, w2a, w2b, ws2)


def _dense3(a, degc, b2, bs2, wpa, wpb, bhp, wc, bc, n, r, nclass):
    return pl.pallas_call(
        functools.partial(_dense3_body, nclass=nclass),
        grid=(n // r,),
        in_specs=[
            _stacked_spec(r, 64, 4),
            _stacked_spec(r, 16),
            _full_spec(b2.shape), _full_spec(bs2.shape),
            _full_spec(wpa.shape), _full_spec(wpb.shape), _full_spec(bhp.shape),
            _full_spec(wc.shape), _full_spec(bc.shape),
        ],
        out_specs=_row_spec(r, 128),
        out_shape=jax.ShapeDtypeStruct((n, 128), jnp.float32),
    )(a, degc, b2, bs2, wpa, wpb, bhp, wc, bc)


def kernel(x, stc_enc, edge_index, batch, W_emb, b_emb, W1, b1, Ws1, bs1,
           W2, b2, Ws2, bs2, Whp_W, Whp_b, Wc, bc):
    n, nf = x.shape
    e = edge_index.shape[1]
    nh = W_emb.shape[1]
    nclass = Wc.shape[1]

    src = edge_index[0].astype(jnp.int32)
    dst = edge_index[1].astype(jnp.int32)

    npad = -(-n // _ROWS) * _ROWS

    dstd = dst.reshape(_NC, _NS, -1, _CHUNK)
    dsta = dst.reshape(_NS, -1, _CHUNK)
    srcx = jnp.concatenate(
        [src + t * npad for t in range(4)]).reshape(4, _NS, -1, _CHUNK)
    init = jnp.concatenate([jnp.ones((1, npad, 16), jnp.float32),
                            jnp.zeros((1, npad, 16), jnp.float32)])

    w1a, w1b = W1[:nf], W1[nf:]
    w2a, w2b = W2[:nh], W2[nh:]
    wpa, wpb = Whp_W[:nh], Whp_W[nh:]
    wc_p = jnp.zeros((nh, 128), jnp.float32).at[:, :nclass].set(Wc)
    bc_p = jnp.zeros((1, 128), jnp.float32).at[:, :nclass].set(bc)
    b_embr = b_emb.reshape(1, nh)
    b1r = b1.reshape(1, nh)
    bs1r = bs1.reshape(1, nh)
    b2r = b2.reshape(1, nh)
    bs2r = bs2.reshape(1, nh)
    bhpr = Whp_b.reshape(1, nh)

    degc = _degree_call(dstd, init)
    g1 = _dense1(x, stc_enc, degc, W_emb, b_embr,
                 w1a, w1b, Ws1, npad, _ROWS)
    a1 = _agg_call(g1.reshape(4 * npad, 64), srcx, dsta)
    g2 = _dense2(a1, degc, b1r, bs1r, w2a, w2b, Ws2,
                 npad, _ROWS)
    a2 = _agg_call(g2.reshape(4 * npad, 64), srcx, dsta)
    out = _dense3(a2, degc, b2r, bs2r, wpa, wpb, bhpr,
                  wc_p, bc_p, npad, _ROWS, nclass)
    return out[:n, :nclass]
